# baseline (speedup 1.0000x reference)
import jax
import jax.numpy as jnp
from jax.experimental import pallas as pl
from jax.experimental.pallas import tpu as pltpu

_BM = 200


def _gcn_body(x_ref, a1_ref, a2_ref, o_ref):
    d = x_ref.shape[1]
    xf = x_ref[...]
    p1 = jnp.dot(a1_ref[...], xf, preferred_element_type=jnp.float32)
    p2 = jnp.dot(a2_ref[...], xf, preferred_element_type=jnp.float32)
    o_ref[:, :d] = p1
    o_ref[:, d:] = p2


@jax.jit
def kernel(x, adj_t, adj_t2):
    n, d = x.shape
    grid = (n // _BM,)
    return pl.pallas_call(
        _gcn_body,
        grid=grid,
        in_specs=[
            pl.BlockSpec((n, d), lambda i: (0, 0)),
            pl.BlockSpec((_BM, n), lambda i: (i, 0)),
            pl.BlockSpec((_BM, n), lambda i: (i, 0)),
        ],
        out_specs=pl.BlockSpec((_BM, 2 * d), lambda i: (i, 0)),
        out_shape=jax.ShapeDtypeStruct((n, 2 * d), jnp.float32),
        compiler_params=pltpu.CompilerParams(
            dimension_semantics=("parallel",),
        ),
    )(x, adj_t, adj_t2)


# speedup vs baseline: 1.0116x; 1.0116x over previous
import jax
import jax.numpy as jnp
from jax.experimental import pallas as pl
from jax.experimental.pallas import tpu as pltpu

_BM = 200


def _gcn_body(x_ref, a1_ref, a2_ref, o_ref):
    d = x_ref.shape[1]
    xf = x_ref[...]
    p1 = jnp.dot(a1_ref[...], xf, preferred_element_type=jnp.float32)
    p2 = jnp.dot(a2_ref[...], xf, preferred_element_type=jnp.float32)
    o_ref[:, :d] = p1
    o_ref[:, d:] = p2


@jax.jit
def kernel(x, adj_t, adj_t2):
    n, d = x.shape
    grid = (n // _BM,)
    return pl.pallas_call(
        _gcn_body,
        grid=grid,
        in_specs=[
            pl.BlockSpec((n, d), lambda i: (0, 0)),
            pl.BlockSpec((_BM, n), lambda i: (i, 0)),
            pl.BlockSpec((_BM, n), lambda i: (i, 0)),
        ],
        out_specs=pl.BlockSpec((_BM, 2 * d), lambda i: (i, 0)),
        out_shape=jax.ShapeDtypeStruct((n, 2 * d), jnp.float32),
        compiler_params=pltpu.CompilerParams(
            dimension_semantics=("arbitrary",),
        ),
    )(x, adj_t, adj_t2)
